# baseline (speedup 1.0000x reference)
import functools

import jax
import jax.numpy as jnp
from jax import lax
from jax.experimental import pallas as pl
from jax.experimental.pallas import tpu as pltpu
from jax.experimental.pallas import tpu_sc as plsc

_B = 16384
_K = 1024
_D = 64
_BETA = 0.25
_BSZ = 4096
_HB = _B // 2
_HGRID = _HB // _BSZ
_DP = 128


def _distance_argmin(z_ref, emb_ref, idx_ref, loss_ref):
    i = pl.program_id(0)
    z = z_ref[...]
    emb = emb_ref[...]
    z2 = jnp.sum(z ** 2, axis=1, keepdims=True)
    e2 = jnp.sum(emb ** 2, axis=1)[None, :]
    ze2 = lax.dot_general(z, emb + emb, (((1,), (1,)), ((), ())),
                          preferred_element_type=jnp.float32)
    dist = z2 + e2 - ze2
    m = jnp.min(dist, axis=1, keepdims=True)
    ids = lax.broadcasted_iota(jnp.int32, (1, _K), 1).astype(jnp.float32)
    idx = jnp.min(jnp.where(dist == m, ids, jnp.float32(_K)), axis=1,
                  keepdims=True)
    idx_ref[...] = idx.astype(jnp.int32)

    @pl.when(i == 0)
    def _():
        loss_ref[...] = jnp.zeros_like(loss_ref)

    loss_ref[...] += jnp.full(loss_ref.shape, jnp.sum(m), dtype=jnp.float32)


def _tc_body_a(z_ref, emb_ref, idx_ref, loss_ref, pad_ref):
    _distance_argmin(z_ref, emb_ref, idx_ref, loss_ref)

    @pl.when(pl.program_id(0) == 0)
    def _():
        pad_ref[...] = jnp.concatenate(
            [emb_ref[...], jnp.zeros((_K, _DP - _D), jnp.float32)], axis=1)


def _make_tc_call(row_off, body, extra_out=(), extra_spec=()):
    return pl.pallas_call(
        body,
        grid=(_HGRID,),
        in_specs=[
            pl.BlockSpec((_BSZ, _D), lambda i: (i + row_off, 0)),
            pl.BlockSpec((_K, _D), lambda i: (0, 0)),
        ],
        out_specs=[
            pl.BlockSpec((_BSZ, 1), lambda i: (i, 0)),
            pl.BlockSpec((8, 128), lambda i: (0, 0)),
            *extra_spec,
        ],
        out_shape=[
            jax.ShapeDtypeStruct((_HB, 1), jnp.int32),
            jax.ShapeDtypeStruct((8, 128), jnp.float32),
            *extra_out,
        ],
    )


_tc_call_a = _make_tc_call(
    0, _tc_body_a,
    extra_out=(jax.ShapeDtypeStruct((_K, _DP), jnp.float32),),
    extra_spec=(pl.BlockSpec((_K, _DP), lambda i: (0, 0)),),
)
_tc_call_b = _make_tc_call(_HGRID, _distance_argmin)


@functools.cache
def _make_sc_gather():
    nc, ns = 2, 16
    nw = nc * ns
    bpw = _HB // nw
    half = bpw // 2
    mesh = plsc.VectorSubcoreMesh(core_axis_name="c", subcore_axis_name="s",
                                  num_cores=nc, num_subcores=ns)

    @functools.partial(
        pl.kernel, mesh=mesh,
        out_type=jax.ShapeDtypeStruct((_HB, _DP), jnp.float32),
        scratch_types=[
            pltpu.VMEM((bpw,), jnp.int32),
            pltpu.VMEM((bpw, _DP), jnp.float32),
            pltpu.SemaphoreType.DMA,
            pltpu.SemaphoreType.DMA,
        ],
    )
    def gather(table_hbm, idx_hbm, out_hbm, idx_v, rows_v, sem0, sem1):
        wid = lax.axis_index("s") * nc + lax.axis_index("c")
        base = wid * bpw
        pltpu.sync_copy(idx_hbm.at[pl.ds(base, bpw)], idx_v)
        c0 = pltpu.async_copy(table_hbm.at[idx_v.at[pl.ds(0, half)]],
                              rows_v.at[pl.ds(0, half)], sem0)
        c1 = pltpu.async_copy(table_hbm.at[idx_v.at[pl.ds(half, half)]],
                              rows_v.at[pl.ds(half, half)], sem1)
        c0.wait()
        pltpu.sync_copy(rows_v.at[pl.ds(0, half)],
                        out_hbm.at[pl.ds(base, half)])
        c1.wait()
        pltpu.sync_copy(rows_v.at[pl.ds(half, half)],
                        out_hbm.at[pl.ds(base + half, half)])

    return gather


def kernel(z_e, emb_weight):
    idx_a2, loss_a, emb_pad = _tc_call_a(z_e, emb_weight)
    idx_b2, loss_b = _tc_call_b(z_e, emb_weight)
    sc = _make_sc_gather()
    zq_a = sc(emb_pad, idx_a2.reshape(_HB))
    zq_b = sc(emb_pad, idx_b2.reshape(_HB))
    z_q_st = jnp.concatenate([zq_a[:, :_D], zq_b[:, :_D]], axis=0)
    idx = jnp.concatenate([idx_a2, idx_b2], axis=0).reshape(_B)
    vq_loss = ((loss_a[0, 0] + loss_b[0, 0])
               * jnp.float32((1.0 + _BETA) / (_B * _D)))
    return (z_q_st, idx, vq_loss)


# speedup vs baseline: 1.1296x; 1.1296x over previous
import functools

import jax
import jax.numpy as jnp
from jax import lax
from jax.experimental import pallas as pl
from jax.experimental.pallas import tpu as pltpu
from jax.experimental.pallas import tpu_sc as plsc

_B = 16384
_K = 1024
_D = 64
_BETA = 0.25
_BSZ = 4096
_GRID = _B // _BSZ
_DP = 128


def _tc_body(z_ref, emb_ref, idx_ref, loss_ref, pad_ref):
    i = pl.program_id(0)
    z = z_ref[...]
    emb = emb_ref[...]
    z2 = jnp.sum(z ** 2, axis=1, keepdims=True)
    e2 = jnp.sum(emb ** 2, axis=1)[None, :]
    ze2 = lax.dot_general(z, emb + emb, (((1,), (1,)), ((), ())),
                          preferred_element_type=jnp.float32)
    dist = z2 + e2 - ze2
    m = jnp.min(dist, axis=1, keepdims=True)
    ids = lax.broadcasted_iota(jnp.int32, (1, _K), 1).astype(jnp.float32)
    idx = jnp.min(jnp.where(dist == m, ids, jnp.float32(_K)), axis=1,
                  keepdims=True)
    idx_ref[...] = idx.astype(jnp.int32)

    @pl.when(i == 0)
    def _():
        loss_ref[...] = jnp.zeros_like(loss_ref)
        pad_ref[...] = jnp.concatenate(
            [emb, jnp.zeros((_K, _DP - _D), jnp.float32)], axis=1)

    msum = lax.dot_general(m, jnp.ones((_BSZ, 1), jnp.float32),
                           (((0,), (0,)), ((), ())),
                           preferred_element_type=jnp.float32)
    loss_ref[...] += jnp.full(loss_ref.shape, msum[0, 0], dtype=jnp.float32)


_tc_call = pl.pallas_call(
    _tc_body,
    grid=(_GRID,),
    in_specs=[
        pl.BlockSpec((_BSZ, _D), lambda i: (i, 0)),
        pl.BlockSpec((_K, _D), lambda i: (0, 0)),
    ],
    out_specs=[
        pl.BlockSpec((_BSZ, 1), lambda i: (i, 0)),
        pl.BlockSpec((8, 128), lambda i: (0, 0)),
        pl.BlockSpec((_K, _DP), lambda i: (0, 0)),
    ],
    out_shape=[
        jax.ShapeDtypeStruct((_B, 1), jnp.int32),
        jax.ShapeDtypeStruct((8, 128), jnp.float32),
        jax.ShapeDtypeStruct((_K, _DP), jnp.float32),
    ],
)


@functools.cache
def _make_sc_gather():
    nc, ns = 2, 16
    nw = nc * ns
    bpw = _B // nw
    nch = 4
    csz = bpw // nch
    mesh = plsc.VectorSubcoreMesh(core_axis_name="c", subcore_axis_name="s",
                                  num_cores=nc, num_subcores=ns)

    @functools.partial(
        pl.kernel, mesh=mesh,
        out_type=jax.ShapeDtypeStruct((_B, _DP), jnp.float32),
        scratch_types=[
            pltpu.VMEM((bpw,), jnp.int32),
            pltpu.VMEM((bpw, _DP), jnp.float32),
        ] + [pltpu.SemaphoreType.DMA] * (2 * nch),
    )
    def gather(table_hbm, idx_hbm, out_hbm, idx_v, rows_v, *sems):
        wid = lax.axis_index("s") * nc + lax.axis_index("c")
        base = wid * bpw
        pltpu.sync_copy(idx_hbm.at[pl.ds(base, bpw)], idx_v)
        gs = [pltpu.async_copy(table_hbm.at[idx_v.at[pl.ds(c * csz, csz)]],
                               rows_v.at[pl.ds(c * csz, csz)], sems[c])
              for c in range(nch)]
        ws = []
        for c in range(nch):
            gs[c].wait()
            ws.append(pltpu.async_copy(
                rows_v.at[pl.ds(c * csz, csz)],
                out_hbm.at[pl.ds(base + c * csz, csz)], sems[nch + c]))
        for w in ws:
            w.wait()

    return gather


def kernel(z_e, emb_weight):
    idx2d, loss_acc, emb_pad = _tc_call(z_e, emb_weight)
    idx = idx2d.reshape(_B)
    z_q_st = _make_sc_gather()(emb_pad, idx)[:, :_D]
    vq_loss = loss_acc[0, 0] * jnp.float32((1.0 + _BETA) / (_B * _D))
    return (z_q_st, idx, vq_loss)
